# baseline (speedup 1.0000x reference)
import jax
import jax.numpy as jnp
from jax import lax
from jax.experimental import pallas as pl
from jax.experimental.pallas import tpu as pltpu
from jax.experimental.pallas import tpu_sc as plsc

NUM_OPS = 100000
EMBED_D = 64
N_NODES = 16384

_info = plsc.get_sparse_core_info()
_NC = _info.num_cores
_NS = _info.num_subcores
_NW = _NC * _NS
_BPW = N_NODES // _NW
_CHUNK = 128
_NCH = _BPW // _CHUNK


def _gather_body(idx_hbm, table_hbm, out_hbm, idx_v, rows_v, sem):
    wid = lax.axis_index("s") * _NC + lax.axis_index("c")
    pltpu.sync_copy(idx_hbm.at[wid], idx_v)
    cps = [
        pltpu.async_copy(table_hbm.at[idx_v.at[j]], rows_v.at[j], sem)
        for j in range(_NCH)
    ]
    for cp in cps:
        cp.wait()
    base = wid * _BPW
    for j in range(_NCH):
        pltpu.sync_copy(rows_v.at[j], out_hbm.at[pl.ds(base + j * _CHUNK, _CHUNK)])


@jax.jit
def kernel(op_ids, table):
    idx = op_ids.astype(jnp.int32).reshape(_NW, _NCH, _CHUNK)
    mesh = plsc.VectorSubcoreMesh(core_axis_name="c", subcore_axis_name="s")
    f = pl.kernel(
        _gather_body,
        out_type=jax.ShapeDtypeStruct((N_NODES, EMBED_D), jnp.float32),
        mesh=mesh,
        scratch_types=[
            pltpu.VMEM((_NCH, _CHUNK), jnp.int32),
            pltpu.VMEM((_NCH, _CHUNK, EMBED_D), jnp.float32),
            pltpu.SemaphoreType.DMA,
        ],
        compiler_params=pltpu.CompilerParams(use_tc_tiling_on_sc=False),
    )
    return f(idx, table)


# speedup vs baseline: 1.0024x; 1.0024x over previous
import jax
import jax.numpy as jnp
from jax import lax
from jax.experimental import pallas as pl
from jax.experimental.pallas import tpu as pltpu
from jax.experimental.pallas import tpu_sc as plsc

NUM_OPS = 100000
EMBED_D = 64
N_NODES = 16384

_info = plsc.get_sparse_core_info()
_NC = _info.num_cores
_NS = _info.num_subcores
_NW = _NC * _NS
_BPW = N_NODES // _NW
_CHUNK = 128
_NCH = _BPW // _CHUNK


def _gather_body(idx_hbm, table_hbm, out_hbm, idx_v, rows_v, gsem0, gsem1,
                 gsem2, gsem3, wsem):
    gsems = (gsem0, gsem1, gsem2, gsem3)
    wid = lax.axis_index("s") * _NC + lax.axis_index("c")
    base = wid * _BPW
    pltpu.sync_copy(idx_hbm.at[pl.ds(base, _BPW)], idx_v)
    gcps = [
        pltpu.async_copy(
            table_hbm.at[idx_v.at[pl.ds(j * _CHUNK, _CHUNK)]],
            rows_v.at[j],
            gsems[j],
        )
        for j in range(_NCH)
    ]
    wcps = []
    for j in range(_NCH):
        gcps[j].wait()
        wcps.append(
            pltpu.async_copy(
                rows_v.at[j], out_hbm.at[pl.ds(base + j * _CHUNK, _CHUNK)], wsem
            )
        )
    for wcp in wcps:
        wcp.wait()


@jax.jit
def kernel(op_ids, table):
    mesh = plsc.VectorSubcoreMesh(core_axis_name="c", subcore_axis_name="s")
    f = pl.kernel(
        _gather_body,
        out_type=jax.ShapeDtypeStruct((N_NODES, EMBED_D), jnp.float32),
        mesh=mesh,
        scratch_types=[
            pltpu.VMEM((_BPW,), jnp.int32),
            pltpu.VMEM((_NCH, _CHUNK, EMBED_D), jnp.float32),
            pltpu.SemaphoreType.DMA,
            pltpu.SemaphoreType.DMA,
            pltpu.SemaphoreType.DMA,
            pltpu.SemaphoreType.DMA,
            pltpu.SemaphoreType.DMA,
        ],
        compiler_params=pltpu.CompilerParams(use_tc_tiling_on_sc=False),
    )
    return f(op_ids.astype(jnp.int32), table)


# speedup vs baseline: 1.9752x; 1.9704x over previous
import jax
import jax.numpy as jnp
from jax import lax
from jax.experimental import pallas as pl
from jax.experimental.pallas import tpu as pltpu
from jax.experimental.pallas import tpu_sc as plsc

NUM_OPS = 100000
EMBED_D = 64
N_NODES = 16384

_info = plsc.get_sparse_core_info()
_NC = _info.num_cores
_NS = _info.num_subcores
_NW = _NC * _NS
_DPW = EMBED_D // _NW
_HALF = N_NODES // 2
_GRP = _HALF // 16


def _gather_body(idx_hbm, table_hbm, out_hbm, row_v, idx_v, out_v, wsem):
    wid = lax.axis_index("s") * _NC + lax.axis_index("c")
    wcps = []
    for d in range(_DPW):
        c = wid * _DPW + d
        pltpu.sync_copy(table_hbm.at[c], row_v)
        for h in range(2):
            pltpu.sync_copy(idx_hbm.at[pl.ds(h * _HALF, _HALF)], idx_v)

            def gbody(g, _):
                iv = idx_v[pl.ds(g * 16, 16)]
                out_v[pl.ds(g * 16, 16)] = plsc.load_gather(row_v, [iv])
                return 0

            lax.fori_loop(0, _GRP, gbody, 0)
            pltpu.sync_copy(out_v, out_hbm.at[c, pl.ds(h * _HALF, _HALF)])


@jax.jit
def kernel(op_ids, table):
    mesh = plsc.VectorSubcoreMesh(core_axis_name="c", subcore_axis_name="s")
    f = pl.kernel(
        _gather_body,
        out_type=jax.ShapeDtypeStruct((EMBED_D, N_NODES), jnp.float32),
        mesh=mesh,
        scratch_types=[
            pltpu.VMEM((NUM_OPS,), jnp.float32),
            pltpu.VMEM((_HALF,), jnp.int32),
            pltpu.VMEM((_HALF,), jnp.float32),
            pltpu.SemaphoreType.DMA,
        ],
        compiler_params=pltpu.CompilerParams(needs_layout_passes=False),
    )
    out_t = f(op_ids.astype(jnp.int32), table.T)
    return out_t.T


# speedup vs baseline: 2.7401x; 1.3872x over previous
import jax
import jax.numpy as jnp
from jax import lax
from jax.experimental import pallas as pl
from jax.experimental.pallas import tpu as pltpu
from jax.experimental.pallas import tpu_sc as plsc

NUM_OPS = 100000
EMBED_D = 64
N_NODES = 16384

_info = plsc.get_sparse_core_info()
_NC = _info.num_cores
_NS = _info.num_subcores
_NW = _NC * _NS
_DPW = EMBED_D // _NW
_Q = N_NODES // 4
_GRP = _Q // 16


def _gather_body(idx_hbm, table_hbm, out_hbm, row_v, idx_v, out_a, out_b,
                 wsem_a, wsem_b):
    wid = lax.axis_index("s") * _NC + lax.axis_index("c")
    pltpu.sync_copy(idx_hbm, idx_v)
    obufs = (out_a, out_b)
    wsems = (wsem_a, wsem_b)
    pending = [None, None]
    for d in range(_DPW):
        c = wid * _DPW + d
        pltpu.sync_copy(table_hbm.at[c], row_v)
        for q in range(4):
            b = q % 2
            if pending[b] is not None:
                pending[b].wait()
            ov = obufs[b]
            qbase = q * _Q

            @plsc.parallel_loop(0, _GRP, unroll=8)
            def _(g):
                iv = idx_v[pl.ds(qbase + g * 16, 16)]
                ov[pl.ds(g * 16, 16)] = plsc.load_gather(row_v, [iv])

            pending[b] = pltpu.async_copy(
                ov, out_hbm.at[c, pl.ds(qbase, _Q)], wsems[b]
            )
    for b in range(2):
        if pending[b] is not None:
            pending[b].wait()


@jax.jit
def kernel(op_ids, table):
    mesh = plsc.VectorSubcoreMesh(core_axis_name="c", subcore_axis_name="s")
    f = pl.kernel(
        _gather_body,
        out_type=jax.ShapeDtypeStruct((EMBED_D, N_NODES), jnp.float32),
        mesh=mesh,
        scratch_types=[
            pltpu.VMEM((NUM_OPS,), jnp.float32),
            pltpu.VMEM((N_NODES,), jnp.int32),
            pltpu.VMEM((_Q,), jnp.float32),
            pltpu.VMEM((_Q,), jnp.float32),
            pltpu.SemaphoreType.DMA,
            pltpu.SemaphoreType.DMA,
        ],
        compiler_params=pltpu.CompilerParams(needs_layout_passes=False),
    )
    out_t = f(op_ids.astype(jnp.int32), table.T)
    return out_t.T


# speedup vs baseline: 2.7455x; 1.0020x over previous
import jax
import jax.numpy as jnp
from jax import lax
from jax.experimental import pallas as pl
from jax.experimental.pallas import tpu as pltpu
from jax.experimental.pallas import tpu_sc as plsc

NUM_OPS = 100000
EMBED_D = 64
N_NODES = 16384

_info = plsc.get_sparse_core_info()
_NC = _info.num_cores
_NS = _info.num_subcores
_NW = _NC * _NS
_DPW = EMBED_D // _NW
_Q = N_NODES // 4
_GRP = _Q // 16


def _gather_body(idx_hbm, table_hbm, out_hbm, row_v, idx_v, out_a, out_b,
                 rsem, wsem_a, wsem_b):
    wid = lax.axis_index("s") * _NC + lax.axis_index("c")
    c0 = wid * _DPW
    rcp = pltpu.async_copy(table_hbm.at[c0], row_v, rsem)
    pltpu.sync_copy(idx_hbm, idx_v)
    obufs = (out_a, out_b)
    wsems = (wsem_a, wsem_b)
    pending = [None, None]
    for d in range(_DPW):
        c = c0 + d
        rcp.wait()
        for q in range(4):
            b = q % 2
            if pending[b] is not None:
                pending[b].wait()
            ov = obufs[b]
            qbase = q * _Q

            @plsc.parallel_loop(0, _GRP, unroll=16)
            def _(g, _ov=ov, _qb=qbase):
                iv = idx_v[pl.ds(_qb + g * 16, 16)]
                _ov[pl.ds(g * 16, 16)] = plsc.load_gather(row_v, [iv])

            if q == 3 and d + 1 < _DPW:
                rcp = pltpu.async_copy(table_hbm.at[c + 1], row_v, rsem)
            pending[b] = pltpu.async_copy(
                ov, out_hbm.at[c, pl.ds(qbase, _Q)], wsems[b]
            )
    for b in range(2):
        if pending[b] is not None:
            pending[b].wait()


@jax.jit
def kernel(op_ids, table):
    mesh = plsc.VectorSubcoreMesh(core_axis_name="c", subcore_axis_name="s")
    f = pl.kernel(
        _gather_body,
        out_type=jax.ShapeDtypeStruct((EMBED_D, N_NODES), jnp.float32),
        mesh=mesh,
        scratch_types=[
            pltpu.VMEM((NUM_OPS,), jnp.float32),
            pltpu.VMEM((N_NODES,), jnp.int32),
            pltpu.VMEM((_Q,), jnp.float32),
            pltpu.VMEM((_Q,), jnp.float32),
            pltpu.SemaphoreType.DMA,
            pltpu.SemaphoreType.DMA,
            pltpu.SemaphoreType.DMA,
        ],
        compiler_params=pltpu.CompilerParams(needs_layout_passes=False),
    )
    out_t = f(op_ids.astype(jnp.int32), table.T)
    return out_t.T
